# baseline (speedup 1.0000x reference)
import functools

import jax
import jax.numpy as jnp
from jax import lax
from jax.experimental import pallas as pl
from jax.experimental.pallas import tpu as pltpu
from jax.experimental.pallas import tpu_sc as plsc

N_USERS = 50000
N_NODES = 100000
DIM = 32
HDIM = DIM // 2
N_LAYERS = 3

NC = 2
NS = 16
LANES = 16
CHUNK = 128
G = 7
G_DEG = 7

ACC2_ROWS = N_NODES + 8
DRAIN = 400
Z_STRIPE = 6400
Z_LAST = N_NODES - (NS - 1) * Z_STRIPE
PREP = 400

_MESH = plsc.VectorSubcoreMesh(core_axis_name="c", subcore_axis_name="s")
_SC_PARAMS = pltpu.CompilerParams(use_tc_tiling_on_sc=False)


def _n_stripe_chunks(tile, chunk_rows):
    return jnp.where(tile < NS - 1, Z_STRIPE // chunk_rows,
                     Z_LAST // chunk_rows)


def _emb_plane_copy(user_hbm, item_hbm, n0, core, buf, write):
    for c in range(NC):
        half = pl.ds(c * HDIM, HDIM)

        @pl.when((core == c) & (n0 < N_USERS))
        def _(half=half):
            sl = user_hbm.at[pl.ds(n0, PREP), half]
            if write:
                pltpu.sync_copy(buf, sl)
            else:
                pltpu.sync_copy(sl, buf)

        @pl.when((core == c) & (n0 >= N_USERS))
        def _(half=half):
            sl = item_hbm.at[pl.ds(n0 - N_USERS, PREP), half]
            if write:
                pltpu.sync_copy(buf, sl)
            else:
                pltpu.sync_copy(sl, buf)


def _sc_degree(col_p):
    total_chunks = col_p.shape[0] // CHUNK
    per_tile = total_chunks // (NC * NS)
    n_groups = per_tile // G_DEG

    @functools.partial(
        pl.kernel,
        out_type=jax.ShapeDtypeStruct((NC, N_NODES, HDIM), jnp.float32),
        mesh=_MESH,
        scratch_types=[
            [pltpu.VMEM((CHUNK,), jnp.int32) for _ in range(G_DEG)],
            pltpu.VMEM((CHUNK, HDIM), jnp.float32),
            pltpu.VMEM((DRAIN, HDIM), jnp.float32),
            pltpu.VMEM_SHARED((ACC2_ROWS, HDIM), jnp.float32),
            pltpu.SemaphoreType.DMA,
        ],
        compiler_params=_SC_PARAMS,
    )
    def k(col_hbm, deg_hbm, colg, ones_v, bounce, acc, ssem):
        core = lax.axis_index("c")
        tile = lax.axis_index("s")
        c0_tile = (core * NS + tile) * per_tile
        n0t = tile * Z_STRIPE

        def ofill(i, carry):
            ones_v[i, pl.ds(0, LANES)] = jnp.ones((LANES,), jnp.float32)
            return carry
        lax.fori_loop(0, CHUNK, ofill, 0)

        def zfill(i, carry):
            bounce[i, pl.ds(0, LANES)] = jnp.zeros((LANES,), jnp.float32)
            return carry
        lax.fori_loop(0, DRAIN, zfill, 0)
        n_b = _n_stripe_chunks(tile, DRAIN)

        def zcopy(i, carry):
            pltpu.sync_copy(bounce, acc.at[pl.ds(n0t + i * DRAIN, DRAIN)])
            return carry
        lax.fori_loop(0, n_b, zcopy, 0)
        plsc.subcore_barrier()

        def group(g, carry):
            e0 = pl.multiple_of((c0_tile + g * G_DEG) * CHUNK, CHUNK)
            for j in range(G_DEG):
                pltpu.sync_copy(col_hbm.at[pl.ds(e0 + j * CHUNK, CHUNK)],
                                colg[j])
            sd = [pltpu.async_copy(ones_v, acc.at[colg[j]], ssem, add=True)
                  for j in range(G_DEG)]
            for d in sd:
                d.wait()
            return carry

        lax.fori_loop(0, n_groups, group, 0)
        plsc.subcore_barrier()

        def dcopy(i, carry):
            n0 = n0t + i * DRAIN
            pltpu.sync_copy(acc.at[pl.ds(n0, DRAIN)], bounce)
            pltpu.sync_copy(bounce, deg_hbm.at[core, pl.ds(n0, DRAIN)])
            return carry
        lax.fori_loop(0, n_b, dcopy, 0)

    return k(col_p)


def _newton_rsqrt(d):
    i = lax.bitcast_convert_type(d, jnp.int32)
    i = jnp.int32(0x5F3759DF) - lax.shift_right_logical(i, 1)
    x = lax.bitcast_convert_type(i, jnp.float32)
    h = d * 0.5
    for _ in range(3):
        x = x * (1.5 - (h * x) * x)
    return jnp.where(d > 0.5, x, 0.0)


def _sc_prep(deg, user_emb, item_emb):
    @functools.partial(
        pl.kernel,
        out_type=(jax.ShapeDtypeStruct((N_NODES, HDIM), jnp.float32),
                  jax.ShapeDtypeStruct((N_NODES, HDIM), jnp.float32),
                  jax.ShapeDtypeStruct((NC, N_NODES, HDIM), jnp.float32)),
        mesh=_MESH,
        scratch_types=[
            pltpu.VMEM((PREP, HDIM), jnp.float32),
            pltpu.VMEM((PREP, HDIM), jnp.float32),
            pltpu.VMEM((PREP, HDIM), jnp.float32),
            pltpu.VMEM((PREP, HDIM), jnp.float32),
            pltpu.SemaphoreType.DMA,
        ],
        compiler_params=_SC_PARAMS,
    )
    def k(deg_hbm, user_hbm, item_hbm, d2x_hbm, sdx_hbm, t1_hbm,
          d0b, dxb, sdb, eb, sem):
        core = lax.axis_index("c")
        tile = lax.axis_index("s")
        n0t = tile * Z_STRIPE
        n_c = _n_stripe_chunks(tile, PREP)

        def chunk(i, carry):
            n0 = n0t + i * PREP
            pltpu.sync_copy(deg_hbm.at[0, pl.ds(n0, PREP)], d0b)
            pltpu.sync_copy(deg_hbm.at[1, pl.ds(n0, PREP)], dxb)
            _emb_plane_copy(user_hbm, item_hbm, n0, core, eb, write=False)

            def work(j, carry2):
                o = pl.ds(0, LANES)
                d = d0b[j, o] + dxb[j, o]
                dv = _newton_rsqrt(d)
                dxb[j, o] = dv * dv
                sdb[j, o] = d * dv
                eb[j, o] = eb[j, o] * dv
                return carry2
            lax.fori_loop(0, PREP, work, 0)

            pltpu.sync_copy(eb, t1_hbm.at[core, pl.ds(n0, PREP)])

            @pl.when(core == 0)
            def _():
                pltpu.sync_copy(dxb, d2x_hbm.at[pl.ds(n0, PREP)])
                pltpu.sync_copy(sdb, sdx_hbm.at[pl.ds(n0, PREP)])
            return carry

        lax.fori_loop(0, n_c, chunk, 0)

    return k(deg, user_emb, item_emb)


def _sc_layer(t, dinv2x, row_p, col_p):
    total_chunks = row_p.shape[0] // CHUNK
    per_tile = total_chunks // NS
    n_groups = per_tile // G

    @functools.partial(
        pl.kernel,
        out_type=jax.ShapeDtypeStruct((NC, N_NODES, HDIM), jnp.float32),
        mesh=_MESH,
        scratch_types=[
            [pltpu.VMEM((CHUNK,), jnp.int32) for _ in range(G)],
            [pltpu.VMEM((CHUNK,), jnp.int32) for _ in range(G)],
            pltpu.VMEM((G, CHUNK, HDIM), jnp.float32),
            pltpu.VMEM((DRAIN, HDIM), jnp.float32),
            pltpu.VMEM((DRAIN, HDIM), jnp.float32),
            pltpu.VMEM_SHARED((ACC2_ROWS, HDIM), jnp.float32),
            pltpu.SemaphoreType.DMA,
            pltpu.SemaphoreType.DMA,
        ],
        compiler_params=_SC_PARAMS,
    )
    def k(t_hbm, d2x_hbm, row_hbm, col_hbm, tn_hbm,
          rowg, colg, bufs, abuf, d2buf, acc, gsem, ssem):
        core = lax.axis_index("c")
        tile = lax.axis_index("s")
        c0_tile = tile * per_tile
        n0t = tile * Z_STRIPE

        def zfill(i, carry):
            abuf[i, pl.ds(0, LANES)] = jnp.zeros((LANES,), jnp.float32)
            return carry
        lax.fori_loop(0, DRAIN, zfill, 0)
        n_b = _n_stripe_chunks(tile, DRAIN)

        def zcopy(i, carry):
            pltpu.sync_copy(abuf, acc.at[pl.ds(n0t + i * DRAIN, DRAIN)])
            return carry
        lax.fori_loop(0, n_b, zcopy, 0)
        plsc.subcore_barrier()

        plane = t_hbm.at[core]

        def group(g, carry):
            e0 = pl.multiple_of((c0_tile + g * G) * CHUNK, CHUNK)
            for j in range(G):
                pltpu.sync_copy(row_hbm.at[pl.ds(e0 + j * CHUNK, CHUNK)],
                                rowg[j])
                pltpu.sync_copy(col_hbm.at[pl.ds(e0 + j * CHUNK, CHUNK)],
                                colg[j])
            gd = [pltpu.async_copy(plane.at[rowg[j]], bufs.at[j], gsem)
                  for j in range(G)]
            sd = []
            for j in range(G):
                gd[j].wait()
                sd.append(pltpu.async_copy(bufs.at[j], acc.at[colg[j]],
                                           ssem, add=True))
            for d in sd:
                d.wait()
            return carry

        lax.fori_loop(0, n_groups, group, 0)
        plsc.subcore_barrier()

        def dcopy(i, carry):
            n0 = n0t + i * DRAIN
            pltpu.sync_copy(acc.at[pl.ds(n0, DRAIN)], abuf)
            pltpu.sync_copy(d2x_hbm.at[pl.ds(n0, DRAIN)], d2buf)

            def scale(j, carry2):
                o = pl.ds(0, LANES)
                abuf[j, o] = abuf[j, o] * d2buf[j, o]
                return carry2
            lax.fori_loop(0, DRAIN, scale, 0)
            pltpu.sync_copy(abuf, tn_hbm.at[core, pl.ds(n0, DRAIN)])
            return carry
        lax.fori_loop(0, n_b, dcopy, 0)

    return k(t, dinv2x, row_p, col_p)


def _sc_final(user_emb, item_emb, t2, t3, t4, sdx):
    @functools.partial(
        pl.kernel,
        out_type=(jax.ShapeDtypeStruct((N_USERS, DIM), jnp.float32),
                  jax.ShapeDtypeStruct((N_NODES - N_USERS, DIM),
                                       jnp.float32)),
        mesh=_MESH,
        scratch_types=[
            pltpu.VMEM((PREP, HDIM), jnp.float32),
            pltpu.VMEM((PREP, HDIM), jnp.float32),
            pltpu.VMEM((PREP, HDIM), jnp.float32),
            pltpu.VMEM((PREP, HDIM), jnp.float32),
            pltpu.VMEM((PREP, HDIM), jnp.float32),
            pltpu.SemaphoreType.DMA,
        ],
        compiler_params=_SC_PARAMS,
    )
    def k(user_hbm, item_hbm, t2_hbm, t3_hbm, t4_hbm, sdx_hbm,
          uout_hbm, iout_hbm, eb, b2, b3, b4, sdb, sem):
        core = lax.axis_index("c")
        tile = lax.axis_index("s")
        n0t = tile * Z_STRIPE
        n_c = _n_stripe_chunks(tile, PREP)

        def chunk(i, carry):
            n0 = n0t + i * PREP
            _emb_plane_copy(user_hbm, item_hbm, n0, core, eb, write=False)
            pltpu.sync_copy(t2_hbm.at[core, pl.ds(n0, PREP)], b2)
            pltpu.sync_copy(t3_hbm.at[core, pl.ds(n0, PREP)], b3)
            pltpu.sync_copy(t4_hbm.at[core, pl.ds(n0, PREP)], b4)
            pltpu.sync_copy(sdx_hbm.at[pl.ds(n0, PREP)], sdb)

            def mix(j, carry2):
                o = pl.ds(0, LANES)
                s = b2[j, o] + b3[j, o] + b4[j, o]
                eb[j, o] = (eb[j, o] + sdb[j, o] * s) * 0.25
                return carry2
            lax.fori_loop(0, PREP, mix, 0)

            _emb_plane_copy(uout_hbm, iout_hbm, n0, core, eb, write=True)
            return carry

        lax.fori_loop(0, n_c, chunk, 0)

    return k(user_emb, item_emb, t2, t3, t4, sdx)


def kernel(edge_index, user_emb, item_emb):
    row = edge_index[0].astype(jnp.int32)
    col = edge_index[1].astype(jnp.int32)

    n_edges = row.shape[0]
    step = NC * NS * CHUNK * G
    e_pad = ((n_edges + step - 1) // step) * step
    pad = e_pad - n_edges
    row_p = jnp.concatenate([row, jnp.zeros((pad,), jnp.int32)])
    col_p = jnp.concatenate([col, jnp.full((pad,), N_NODES, jnp.int32)])

    deg = _sc_degree(col_p)
    d2x, sdx, t = _sc_prep(deg, user_emb, item_emb)
    t2 = _sc_layer(t, d2x, row_p, col_p)
    t3 = _sc_layer(t2, d2x, row_p, col_p)
    t4 = _sc_layer(t3, d2x, row_p, col_p)
    return _sc_final(user_emb, item_emb, t2, t3, t4, sdx)


# speedup vs baseline: 2.0472x; 2.0472x over previous
import functools

import jax
import jax.numpy as jnp
from jax import lax
from jax.experimental import pallas as pl
from jax.experimental.pallas import tpu as pltpu
from jax.experimental.pallas import tpu_sc as plsc

N_USERS = 50000
N_NODES = 100000
DIM = 32
HDIM = DIM // 2
N_LAYERS = 3

NC = 2
NS = 16
LANES = 16
CHUNK = 128
G = 7
G_DEG = 7

ACC2_ROWS = N_NODES + 8
DRAIN = 400
Z_STRIPE = 6400
Z_LAST = N_NODES - (NS - 1) * Z_STRIPE
PREP = 400

_MESH = plsc.VectorSubcoreMesh(core_axis_name="c", subcore_axis_name="s")
_SC_PARAMS = pltpu.CompilerParams(use_tc_tiling_on_sc=False)


def _n_stripe_chunks(tile, chunk_rows):
    return jnp.where(tile < NS - 1, Z_STRIPE // chunk_rows,
                     Z_LAST // chunk_rows)


def _emb_plane_copy(user_hbm, item_hbm, n0, core, buf, write):
    for c in range(NC):
        half = pl.ds(c * HDIM, HDIM)

        @pl.when((core == c) & (n0 < N_USERS))
        def _(half=half):
            sl = user_hbm.at[pl.ds(n0, PREP), half]
            if write:
                pltpu.sync_copy(buf, sl)
            else:
                pltpu.sync_copy(sl, buf)

        @pl.when((core == c) & (n0 >= N_USERS))
        def _(half=half):
            sl = item_hbm.at[pl.ds(n0 - N_USERS, PREP), half]
            if write:
                pltpu.sync_copy(buf, sl)
            else:
                pltpu.sync_copy(sl, buf)


def _sc_degree(col_p):
    total_chunks = col_p.shape[0] // CHUNK
    per_tile = total_chunks // (NC * NS)
    n_groups = per_tile // G_DEG

    @functools.partial(
        pl.kernel,
        out_type=jax.ShapeDtypeStruct((NC, N_NODES, HDIM), jnp.float32),
        mesh=_MESH,
        scratch_types=[
            pltpu.VMEM((G_DEG * CHUNK,), jnp.int32),
            [pltpu.VMEM((CHUNK,), jnp.int32) for _ in range(G_DEG)],
            pltpu.VMEM((CHUNK, HDIM), jnp.float32),
            pltpu.VMEM((DRAIN, HDIM), jnp.float32),
            pltpu.VMEM_SHARED((ACC2_ROWS, HDIM), jnp.float32),
            pltpu.SemaphoreType.DMA,
        ],
        compiler_params=_SC_PARAMS,
    )
    def k(col_hbm, deg_hbm, colg1d, colg, ones_v, bounce, acc, ssem):
        core = lax.axis_index("c")
        tile = lax.axis_index("s")
        c0_tile = (core * NS + tile) * per_tile
        n0t = tile * Z_STRIPE

        def ofill(i, carry):
            ones_v[i, pl.ds(0, LANES)] = jnp.ones((LANES,), jnp.float32)
            return carry
        lax.fori_loop(0, CHUNK, ofill, 0)

        def zfill(i, carry):
            bounce[i, pl.ds(0, LANES)] = jnp.zeros((LANES,), jnp.float32)
            return carry
        lax.fori_loop(0, DRAIN, zfill, 0)
        n_b = _n_stripe_chunks(tile, DRAIN)

        def zcopy(i, carry):
            pltpu.sync_copy(bounce, acc.at[pl.ds(n0t + i * DRAIN, DRAIN)])
            return carry
        lax.fori_loop(0, n_b, zcopy, 0)
        plsc.subcore_barrier()

        def group(g, carry):
            e0 = pl.multiple_of((c0_tile + g * G_DEG) * CHUNK, CHUNK)
            pltpu.sync_copy(col_hbm.at[pl.ds(e0, G_DEG * CHUNK)], colg1d)
            for j in range(G_DEG):
                for kk in range(CHUNK // LANES):
                    colg[j][pl.ds(kk * LANES, LANES)] = (
                        colg1d[pl.ds(j * CHUNK + kk * LANES, LANES)])
            sd = [pltpu.async_copy(ones_v, acc.at[colg[j]], ssem, add=True)
                  for j in range(G_DEG)]
            for d in sd:
                d.wait()
            return carry

        lax.fori_loop(0, n_groups, group, 0)
        plsc.subcore_barrier()

        def dcopy(i, carry):
            n0 = n0t + i * DRAIN
            pltpu.sync_copy(acc.at[pl.ds(n0, DRAIN)], bounce)
            pltpu.sync_copy(bounce, deg_hbm.at[core, pl.ds(n0, DRAIN)])
            return carry
        lax.fori_loop(0, n_b, dcopy, 0)

    return k(col_p)


def _newton_rsqrt(d):
    i = lax.bitcast_convert_type(d, jnp.int32)
    i = jnp.int32(0x5F3759DF) - lax.shift_right_logical(i, 1)
    x = lax.bitcast_convert_type(i, jnp.float32)
    h = d * 0.5
    for _ in range(3):
        x = x * (1.5 - (h * x) * x)
    return jnp.where(d > 0.5, x, 0.0)


def _sc_prep(deg, user_emb, item_emb):
    @functools.partial(
        pl.kernel,
        out_type=(jax.ShapeDtypeStruct((N_NODES, HDIM), jnp.float32),
                  jax.ShapeDtypeStruct((N_NODES, HDIM), jnp.float32),
                  jax.ShapeDtypeStruct((NC, N_NODES, HDIM), jnp.float32)),
        mesh=_MESH,
        scratch_types=[
            pltpu.VMEM((PREP, HDIM), jnp.float32),
            pltpu.VMEM((PREP, HDIM), jnp.float32),
            pltpu.VMEM((PREP, HDIM), jnp.float32),
            pltpu.VMEM((PREP, HDIM), jnp.float32),
            pltpu.SemaphoreType.DMA,
        ],
        compiler_params=_SC_PARAMS,
    )
    def k(deg_hbm, user_hbm, item_hbm, d2x_hbm, sdx_hbm, t1_hbm,
          d0b, dxb, sdb, eb, sem):
        core = lax.axis_index("c")
        tile = lax.axis_index("s")
        n0t = tile * Z_STRIPE
        n_c = _n_stripe_chunks(tile, PREP)

        def chunk(i, carry):
            n0 = n0t + i * PREP
            pltpu.sync_copy(deg_hbm.at[0, pl.ds(n0, PREP)], d0b)
            pltpu.sync_copy(deg_hbm.at[1, pl.ds(n0, PREP)], dxb)
            _emb_plane_copy(user_hbm, item_hbm, n0, core, eb, write=False)

            def work(j, carry2):
                o = pl.ds(0, LANES)
                d = d0b[j, o] + dxb[j, o]
                dv = _newton_rsqrt(d)
                dxb[j, o] = dv * dv
                sdb[j, o] = d * dv
                eb[j, o] = eb[j, o] * dv
                return carry2
            lax.fori_loop(0, PREP, work, 0)

            pltpu.sync_copy(eb, t1_hbm.at[core, pl.ds(n0, PREP)])

            @pl.when(core == 0)
            def _():
                pltpu.sync_copy(dxb, d2x_hbm.at[pl.ds(n0, PREP)])
                pltpu.sync_copy(sdb, sdx_hbm.at[pl.ds(n0, PREP)])
            return carry

        lax.fori_loop(0, n_c, chunk, 0)

    return k(deg, user_emb, item_emb)


def _sc_layer(t, dinv2x, row_p, col_p):
    total_chunks = row_p.shape[0] // CHUNK
    per_tile = total_chunks // NS
    n_groups = per_tile // G

    @functools.partial(
        pl.kernel,
        out_type=jax.ShapeDtypeStruct((NC, N_NODES, HDIM), jnp.float32),
        mesh=_MESH,
        scratch_types=[
            pltpu.VMEM((G * CHUNK,), jnp.int32),
            pltpu.VMEM((G * CHUNK,), jnp.int32),
            [pltpu.VMEM((CHUNK,), jnp.int32) for _ in range(G)],
            pltpu.VMEM((G, CHUNK, HDIM), jnp.float32),
            pltpu.VMEM((DRAIN, HDIM), jnp.float32),
            pltpu.VMEM((DRAIN, HDIM), jnp.float32),
            pltpu.VMEM_SHARED((ACC2_ROWS, HDIM), jnp.float32),
            pltpu.SemaphoreType.DMA,
            pltpu.SemaphoreType.DMA,
        ],
        compiler_params=_SC_PARAMS,
    )
    def k(t_hbm, d2x_hbm, row_hbm, col_hbm, tn_hbm,
          rowg1d, colg1d, colg, bufs, abuf, d2buf, acc, gsem, ssem):
        core = lax.axis_index("c")
        tile = lax.axis_index("s")
        c0_tile = tile * per_tile
        n0t = tile * Z_STRIPE

        def zfill(i, carry):
            abuf[i, pl.ds(0, LANES)] = jnp.zeros((LANES,), jnp.float32)
            return carry
        lax.fori_loop(0, DRAIN, zfill, 0)
        n_b = _n_stripe_chunks(tile, DRAIN)

        def zcopy(i, carry):
            pltpu.sync_copy(abuf, acc.at[pl.ds(n0t + i * DRAIN, DRAIN)])
            return carry
        lax.fori_loop(0, n_b, zcopy, 0)
        plsc.subcore_barrier()

        plane = t_hbm.at[core]

        def group(g, carry):
            e0 = pl.multiple_of((c0_tile + g * G) * CHUNK, CHUNK)
            pltpu.sync_copy(row_hbm.at[pl.ds(e0, G * CHUNK)], rowg1d)
            pltpu.sync_copy(col_hbm.at[pl.ds(e0, G * CHUNK)], colg1d)
            for j in range(G):
                for kk in range(CHUNK // LANES):
                    colg[j][pl.ds(kk * LANES, LANES)] = (
                        colg1d[pl.ds(j * CHUNK + kk * LANES, LANES)])
            gd = [pltpu.async_copy(
                      plane.at[rowg1d.at[pl.ds(j * CHUNK, CHUNK)]],
                      bufs.at[j], gsem)
                  for j in range(G)]
            sd = []
            for j in range(G):
                gd[j].wait()
                sd.append(pltpu.async_copy(bufs.at[j], acc.at[colg[j]],
                                           ssem, add=True))
            for d in sd:
                d.wait()
            return carry

        lax.fori_loop(0, n_groups, group, 0)
        plsc.subcore_barrier()

        def dcopy(i, carry):
            n0 = n0t + i * DRAIN
            pltpu.sync_copy(acc.at[pl.ds(n0, DRAIN)], abuf)
            pltpu.sync_copy(d2x_hbm.at[pl.ds(n0, DRAIN)], d2buf)

            def scale(j, carry2):
                o = pl.ds(0, LANES)
                abuf[j, o] = abuf[j, o] * d2buf[j, o]
                return carry2
            lax.fori_loop(0, DRAIN, scale, 0)
            pltpu.sync_copy(abuf, tn_hbm.at[core, pl.ds(n0, DRAIN)])
            return carry
        lax.fori_loop(0, n_b, dcopy, 0)

    return k(t, dinv2x, row_p, col_p)


def _sc_final(user_emb, item_emb, t2, t3, t4, sdx):
    @functools.partial(
        pl.kernel,
        out_type=(jax.ShapeDtypeStruct((N_USERS, DIM), jnp.float32),
                  jax.ShapeDtypeStruct((N_NODES - N_USERS, DIM),
                                       jnp.float32)),
        mesh=_MESH,
        scratch_types=[
            pltpu.VMEM((PREP, HDIM), jnp.float32),
            pltpu.VMEM((PREP, HDIM), jnp.float32),
            pltpu.VMEM((PREP, HDIM), jnp.float32),
            pltpu.VMEM((PREP, HDIM), jnp.float32),
            pltpu.VMEM((PREP, HDIM), jnp.float32),
            pltpu.SemaphoreType.DMA,
        ],
        compiler_params=_SC_PARAMS,
    )
    def k(user_hbm, item_hbm, t2_hbm, t3_hbm, t4_hbm, sdx_hbm,
          uout_hbm, iout_hbm, eb, b2, b3, b4, sdb, sem):
        core = lax.axis_index("c")
        tile = lax.axis_index("s")
        n0t = tile * Z_STRIPE
        n_c = _n_stripe_chunks(tile, PREP)

        def chunk(i, carry):
            n0 = n0t + i * PREP
            _emb_plane_copy(user_hbm, item_hbm, n0, core, eb, write=False)
            pltpu.sync_copy(t2_hbm.at[core, pl.ds(n0, PREP)], b2)
            pltpu.sync_copy(t3_hbm.at[core, pl.ds(n0, PREP)], b3)
            pltpu.sync_copy(t4_hbm.at[core, pl.ds(n0, PREP)], b4)
            pltpu.sync_copy(sdx_hbm.at[pl.ds(n0, PREP)], sdb)

            def mix(j, carry2):
                o = pl.ds(0, LANES)
                s = b2[j, o] + b3[j, o] + b4[j, o]
                eb[j, o] = (eb[j, o] + sdb[j, o] * s) * 0.25
                return carry2
            lax.fori_loop(0, PREP, mix, 0)

            _emb_plane_copy(uout_hbm, iout_hbm, n0, core, eb, write=True)
            return carry

        lax.fori_loop(0, n_c, chunk, 0)

    return k(user_emb, item_emb, t2, t3, t4, sdx)


def kernel(edge_index, user_emb, item_emb):
    row = edge_index[0].astype(jnp.int32)
    col = edge_index[1].astype(jnp.int32)

    n_edges = row.shape[0]
    step = NC * NS * CHUNK * G
    e_pad = ((n_edges + step - 1) // step) * step
    pad = e_pad - n_edges
    row_p = jnp.concatenate([row, jnp.zeros((pad,), jnp.int32)])
    col_p = jnp.concatenate([col, jnp.full((pad,), N_NODES, jnp.int32)])

    deg = _sc_degree(col_p)
    d2x, sdx, t = _sc_prep(deg, user_emb, item_emb)
    t2 = _sc_layer(t, d2x, row_p, col_p)
    t3 = _sc_layer(t2, d2x, row_p, col_p)
    t4 = _sc_layer(t3, d2x, row_p, col_p)
    return _sc_final(user_emb, item_emb, t2, t3, t4, sdx)


# speedup vs baseline: 2.0843x; 1.0181x over previous
import functools

import jax
import jax.numpy as jnp
from jax import lax
from jax.experimental import pallas as pl
from jax.experimental.pallas import tpu as pltpu
from jax.experimental.pallas import tpu_sc as plsc

N_USERS = 50000
N_NODES = 100000
DIM = 32
HDIM = DIM // 2
N_LAYERS = 3

NC = 2
NS = 16
LANES = 16
CHUNK = 128
G = 8
G_DEG = 8

ACC2_ROWS = N_NODES + 8
DRAIN = 200
Z_STRIPE = 6400
Z_LAST = N_NODES - (NS - 1) * Z_STRIPE
PREP = 400

_MESH = plsc.VectorSubcoreMesh(core_axis_name="c", subcore_axis_name="s")
_SC_PARAMS = pltpu.CompilerParams(use_tc_tiling_on_sc=False)


def _n_stripe_chunks(tile, chunk_rows):
    return jnp.where(tile < NS - 1, Z_STRIPE // chunk_rows,
                     Z_LAST // chunk_rows)


def _emb_plane_copy(user_hbm, item_hbm, n0, core, buf, write):
    for c in range(NC):
        half = pl.ds(c * HDIM, HDIM)

        @pl.when((core == c) & (n0 < N_USERS))
        def _(half=half):
            sl = user_hbm.at[pl.ds(n0, PREP), half]
            if write:
                pltpu.sync_copy(buf, sl)
            else:
                pltpu.sync_copy(sl, buf)

        @pl.when((core == c) & (n0 >= N_USERS))
        def _(half=half):
            sl = item_hbm.at[pl.ds(n0 - N_USERS, PREP), half]
            if write:
                pltpu.sync_copy(buf, sl)
            else:
                pltpu.sync_copy(sl, buf)


def _sc_degree(col_p):
    total_chunks = col_p.shape[0] // CHUNK
    per_tile = total_chunks // (NC * NS)
    n_groups = per_tile // G_DEG

    @functools.partial(
        pl.kernel,
        out_type=jax.ShapeDtypeStruct((NC, N_NODES, HDIM), jnp.float32),
        mesh=_MESH,
        scratch_types=[
            pltpu.VMEM((G_DEG * CHUNK,), jnp.int32),
            [pltpu.VMEM((CHUNK,), jnp.int32) for _ in range(G_DEG)],
            pltpu.VMEM((CHUNK, HDIM), jnp.float32),
            pltpu.VMEM((DRAIN, HDIM), jnp.float32),
            pltpu.VMEM_SHARED((ACC2_ROWS, HDIM), jnp.float32),
            pltpu.SemaphoreType.DMA,
        ],
        compiler_params=_SC_PARAMS,
    )
    def k(col_hbm, deg_hbm, colg1d, colg, ones_v, bounce, acc, ssem):
        core = lax.axis_index("c")
        tile = lax.axis_index("s")
        c0_tile = (core * NS + tile) * per_tile
        n0t = tile * Z_STRIPE

        def ofill(i, carry):
            ones_v[i, pl.ds(0, LANES)] = jnp.ones((LANES,), jnp.float32)
            return carry
        lax.fori_loop(0, CHUNK, ofill, 0)

        def zfill(i, carry):
            bounce[i, pl.ds(0, LANES)] = jnp.zeros((LANES,), jnp.float32)
            return carry
        lax.fori_loop(0, DRAIN, zfill, 0)
        n_b = _n_stripe_chunks(tile, DRAIN)

        def zcopy(i, carry):
            pltpu.sync_copy(bounce, acc.at[pl.ds(n0t + i * DRAIN, DRAIN)])
            return carry
        lax.fori_loop(0, n_b, zcopy, 0)
        plsc.subcore_barrier()

        def group(g, carry):
            e0 = pl.multiple_of((c0_tile + g * G_DEG) * CHUNK, CHUNK)
            pltpu.sync_copy(col_hbm.at[pl.ds(e0, G_DEG * CHUNK)], colg1d)
            for j in range(G_DEG):
                for kk in range(CHUNK // LANES):
                    colg[j][pl.ds(kk * LANES, LANES)] = (
                        colg1d[pl.ds(j * CHUNK + kk * LANES, LANES)])
            sd = [pltpu.async_copy(ones_v, acc.at[colg[j]], ssem, add=True)
                  for j in range(G_DEG)]
            for d in sd:
                d.wait()
            return carry

        lax.fori_loop(0, n_groups, group, 0)
        plsc.subcore_barrier()

        def dcopy(i, carry):
            n0 = n0t + i * DRAIN
            pltpu.sync_copy(acc.at[pl.ds(n0, DRAIN)], bounce)
            pltpu.sync_copy(bounce, deg_hbm.at[core, pl.ds(n0, DRAIN)])
            return carry
        lax.fori_loop(0, n_b, dcopy, 0)

    return k(col_p)


def _newton_rsqrt(d):
    i = lax.bitcast_convert_type(d, jnp.int32)
    i = jnp.int32(0x5F3759DF) - lax.shift_right_logical(i, 1)
    x = lax.bitcast_convert_type(i, jnp.float32)
    h = d * 0.5
    for _ in range(3):
        x = x * (1.5 - (h * x) * x)
    return jnp.where(d > 0.5, x, 0.0)


def _sc_prep(deg, user_emb, item_emb):
    @functools.partial(
        pl.kernel,
        out_type=(jax.ShapeDtypeStruct((N_NODES, HDIM), jnp.float32),
                  jax.ShapeDtypeStruct((N_NODES, HDIM), jnp.float32),
                  jax.ShapeDtypeStruct((NC, N_NODES, HDIM), jnp.float32)),
        mesh=_MESH,
        scratch_types=[
            pltpu.VMEM((PREP, HDIM), jnp.float32),
            pltpu.VMEM((PREP, HDIM), jnp.float32),
            pltpu.VMEM((PREP, HDIM), jnp.float32),
            pltpu.VMEM((PREP, HDIM), jnp.float32),
            pltpu.SemaphoreType.DMA,
        ],
        compiler_params=_SC_PARAMS,
    )
    def k(deg_hbm, user_hbm, item_hbm, d2x_hbm, sdx_hbm, t1_hbm,
          d0b, dxb, sdb, eb, sem):
        core = lax.axis_index("c")
        tile = lax.axis_index("s")
        n0t = tile * Z_STRIPE
        n_c = _n_stripe_chunks(tile, PREP)

        def chunk(i, carry):
            n0 = n0t + i * PREP
            pltpu.sync_copy(deg_hbm.at[0, pl.ds(n0, PREP)], d0b)
            pltpu.sync_copy(deg_hbm.at[1, pl.ds(n0, PREP)], dxb)
            _emb_plane_copy(user_hbm, item_hbm, n0, core, eb, write=False)

            def work(j, carry2):
                o = pl.ds(0, LANES)
                d = d0b[j, o] + dxb[j, o]
                dv = _newton_rsqrt(d)
                dxb[j, o] = dv * dv
                sdb[j, o] = d * dv
                eb[j, o] = eb[j, o] * dv
                return carry2
            lax.fori_loop(0, PREP, work, 0)

            pltpu.sync_copy(eb, t1_hbm.at[core, pl.ds(n0, PREP)])

            @pl.when(core == 0)
            def _():
                pltpu.sync_copy(dxb, d2x_hbm.at[pl.ds(n0, PREP)])
                pltpu.sync_copy(sdb, sdx_hbm.at[pl.ds(n0, PREP)])
            return carry

        lax.fori_loop(0, n_c, chunk, 0)

    return k(deg, user_emb, item_emb)


def _sc_layer(t, dinv2x, row_p, col_p):
    total_chunks = row_p.shape[0] // CHUNK
    per_tile = total_chunks // NS
    n_groups = per_tile // G

    @functools.partial(
        pl.kernel,
        out_type=jax.ShapeDtypeStruct((NC, N_NODES, HDIM), jnp.float32),
        mesh=_MESH,
        scratch_types=[
            pltpu.VMEM((G * CHUNK,), jnp.int32),
            pltpu.VMEM((G * CHUNK,), jnp.int32),
            [pltpu.VMEM((CHUNK,), jnp.int32) for _ in range(G)],
            pltpu.VMEM((G, CHUNK, HDIM), jnp.float32),
            pltpu.VMEM((DRAIN, HDIM), jnp.float32),
            pltpu.VMEM((DRAIN, HDIM), jnp.float32),
            pltpu.VMEM_SHARED((ACC2_ROWS, HDIM), jnp.float32),
            pltpu.SemaphoreType.DMA,
            pltpu.SemaphoreType.DMA,
        ],
        compiler_params=_SC_PARAMS,
    )
    def k(t_hbm, d2x_hbm, row_hbm, col_hbm, tn_hbm,
          rowg1d, colg1d, colg, bufs, abuf, d2buf, acc, gsem, ssem):
        core = lax.axis_index("c")
        tile = lax.axis_index("s")
        c0_tile = tile * per_tile
        n0t = tile * Z_STRIPE

        def zfill(i, carry):
            abuf[i, pl.ds(0, LANES)] = jnp.zeros((LANES,), jnp.float32)
            return carry
        lax.fori_loop(0, DRAIN, zfill, 0)
        n_b = _n_stripe_chunks(tile, DRAIN)

        def zcopy(i, carry):
            pltpu.sync_copy(abuf, acc.at[pl.ds(n0t + i * DRAIN, DRAIN)])
            return carry
        lax.fori_loop(0, n_b, zcopy, 0)
        plsc.subcore_barrier()

        plane = t_hbm.at[core]

        def group(g, carry):
            e0 = pl.multiple_of((c0_tile + g * G) * CHUNK, CHUNK)
            pltpu.sync_copy(row_hbm.at[pl.ds(e0, G * CHUNK)], rowg1d)
            pltpu.sync_copy(col_hbm.at[pl.ds(e0, G * CHUNK)], colg1d)
            for j in range(G):
                for kk in range(CHUNK // LANES):
                    colg[j][pl.ds(kk * LANES, LANES)] = (
                        colg1d[pl.ds(j * CHUNK + kk * LANES, LANES)])
            gd = [pltpu.async_copy(
                      plane.at[rowg1d.at[pl.ds(j * CHUNK, CHUNK)]],
                      bufs.at[j], gsem)
                  for j in range(G)]
            sd = []
            for j in range(G):
                gd[j].wait()
                sd.append(pltpu.async_copy(bufs.at[j], acc.at[colg[j]],
                                           ssem, add=True))
            for d in sd:
                d.wait()
            return carry

        lax.fori_loop(0, n_groups, group, 0)
        plsc.subcore_barrier()

        def dcopy(i, carry):
            n0 = n0t + i * DRAIN
            pltpu.sync_copy(acc.at[pl.ds(n0, DRAIN)], abuf)
            pltpu.sync_copy(d2x_hbm.at[pl.ds(n0, DRAIN)], d2buf)

            def scale(j, carry2):
                o = pl.ds(0, LANES)
                abuf[j, o] = abuf[j, o] * d2buf[j, o]
                return carry2
            lax.fori_loop(0, DRAIN, scale, 0)
            pltpu.sync_copy(abuf, tn_hbm.at[core, pl.ds(n0, DRAIN)])
            return carry
        lax.fori_loop(0, n_b, dcopy, 0)

    return k(t, dinv2x, row_p, col_p)


def _sc_final(user_emb, item_emb, t2, t3, t4, sdx):
    @functools.partial(
        pl.kernel,
        out_type=(jax.ShapeDtypeStruct((N_USERS, DIM), jnp.float32),
                  jax.ShapeDtypeStruct((N_NODES - N_USERS, DIM),
                                       jnp.float32)),
        mesh=_MESH,
        scratch_types=[
            pltpu.VMEM((PREP, HDIM), jnp.float32),
            pltpu.VMEM((PREP, HDIM), jnp.float32),
            pltpu.VMEM((PREP, HDIM), jnp.float32),
            pltpu.VMEM((PREP, HDIM), jnp.float32),
            pltpu.VMEM((PREP, HDIM), jnp.float32),
            pltpu.SemaphoreType.DMA,
        ],
        compiler_params=_SC_PARAMS,
    )
    def k(user_hbm, item_hbm, t2_hbm, t3_hbm, t4_hbm, sdx_hbm,
          uout_hbm, iout_hbm, eb, b2, b3, b4, sdb, sem):
        core = lax.axis_index("c")
        tile = lax.axis_index("s")
        n0t = tile * Z_STRIPE
        n_c = _n_stripe_chunks(tile, PREP)

        def chunk(i, carry):
            n0 = n0t + i * PREP
            _emb_plane_copy(user_hbm, item_hbm, n0, core, eb, write=False)
            pltpu.sync_copy(t2_hbm.at[core, pl.ds(n0, PREP)], b2)
            pltpu.sync_copy(t3_hbm.at[core, pl.ds(n0, PREP)], b3)
            pltpu.sync_copy(t4_hbm.at[core, pl.ds(n0, PREP)], b4)
            pltpu.sync_copy(sdx_hbm.at[pl.ds(n0, PREP)], sdb)

            def mix(j, carry2):
                o = pl.ds(0, LANES)
                s = b2[j, o] + b3[j, o] + b4[j, o]
                eb[j, o] = (eb[j, o] + sdb[j, o] * s) * 0.25
                return carry2
            lax.fori_loop(0, PREP, mix, 0)

            _emb_plane_copy(uout_hbm, iout_hbm, n0, core, eb, write=True)
            return carry

        lax.fori_loop(0, n_c, chunk, 0)

    return k(user_emb, item_emb, t2, t3, t4, sdx)


def kernel(edge_index, user_emb, item_emb):
    row = edge_index[0].astype(jnp.int32)
    col = edge_index[1].astype(jnp.int32)

    n_edges = row.shape[0]
    step = NC * NS * CHUNK * G
    e_pad = ((n_edges + step - 1) // step) * step
    pad = e_pad - n_edges
    row_p = jnp.concatenate([row, jnp.zeros((pad,), jnp.int32)])
    col_p = jnp.concatenate([col, jnp.full((pad,), N_NODES, jnp.int32)])

    deg = _sc_degree(col_p)
    d2x, sdx, t = _sc_prep(deg, user_emb, item_emb)
    t2 = _sc_layer(t, d2x, row_p, col_p)
    t3 = _sc_layer(t2, d2x, row_p, col_p)
    t4 = _sc_layer(t3, d2x, row_p, col_p)
    return _sc_final(user_emb, item_emb, t2, t3, t4, sdx)
